# baseline (speedup 1.0000x reference)
import jax
import jax.numpy as jnp
from jax import lax
from jax.experimental import pallas as pl

B = 16384
C = 1001
NADV = 1000

R = 512


def _tc_body(x_ref, a_ref, o_ref):
    xb = x_ref[...]
    av = a_ref[...]
    cols = lax.broadcasted_iota(jnp.int32, (R, C), 1)
    sel = jnp.where(cols == av, xb, 0.0)
    wcol = lax.broadcasted_iota(jnp.int32, (C, 1), 0)
    s = jnp.float32(1.0 / NADV)
    w = jnp.where(wcol == NADV, jnp.float32(1.0), -s)
    ones = jnp.full((C, 1), jnp.float32(1.0))
    d = jnp.dot(xb, w, preferred_element_type=jnp.float32)
    g = jnp.dot(sel, ones, preferred_element_type=jnp.float32)
    o_ref[...] = d + g


def kernel(x, a):
    a32 = a.astype(jnp.int32)
    out = pl.pallas_call(
        _tc_body,
        grid=(B // R,),
        in_specs=[
            pl.BlockSpec((R, C), lambda i: (i, 0)),
            pl.BlockSpec((R, 1), lambda i: (i, 0)),
        ],
        out_specs=pl.BlockSpec((R, 1), lambda i: (i, 0)),
        out_shape=jax.ShapeDtypeStruct((B, 1), jnp.float32),
    )(x, a32)
    return out
